# Initial kernel scaffold; baseline (speedup 1.0000x reference)
import jax
import jax.numpy as jnp
from jax.experimental import pallas as pl


def kernel(x, router_W, router_b, W1, b1, W2, b2):
    raise NotImplementedError("write your pallas kernel here")


# speedup vs baseline: 2.7321x; 2.7321x over previous
import functools

import jax
import jax.numpy as jnp
from jax import lax
from jax.experimental import pallas as pl
from jax.experimental.pallas import tpu as pltpu
import jax.experimental.pallas.tpu_sc as plsc

N = 2048
D = 768
E = 8
F = 3072
K = 2
P2 = 2 * N
BLK = 128
NB = P2 // BLK + E
P = NB * BLK

NC = 2
NS = 16
NW = NC * NS
LANES = 16

@functools.cache
def _sc_mesh():
    return plsc.VectorSubcoreMesh(
        core_axis_name="c", subcore_axis_name="s",
        num_cores=NC, num_subcores=NS)


def _router_body(rwT_ref, rb_ref, xT_ref, wpair_ref, slot_ref, be_ref, aux_ref):
    logits = jnp.dot(rwT_ref[...], xT_ref[...],
                     preferred_element_type=jnp.float32) + rb_ref[...]
    m = jnp.max(logits, axis=0, keepdims=True)
    ex = jnp.exp(logits - m)
    probs = ex / jnp.sum(ex, axis=0, keepdims=True)

    mp = jnp.sum(probs, axis=1, keepdims=True) * (1.0 / N)
    mu = jnp.sum(mp) * (1.0 / E)
    aux_ref[0, 0] = jnp.sum((mp - mu) ** 2) * (1.0 / (E - 1))

    eid = lax.broadcasted_iota(jnp.int32, (E, N), 0)
    p1 = jnp.max(probs, axis=0, keepdims=True)
    i1 = jnp.min(jnp.where(probs == p1, eid, E), axis=0, keepdims=True)
    oh1 = eid == i1
    masked = jnp.where(oh1, -1.0, probs)
    p2 = jnp.max(masked, axis=0, keepdims=True)
    i2 = jnp.min(jnp.where(masked == p2, eid, E), axis=0, keepdims=True)
    oh2 = eid == i2
    sw = p1 + p2
    wpair_ref[...] = jnp.concatenate([p1 / sw, p2 / sw], axis=1)

    oh = jnp.concatenate([oh1, oh2], axis=1).astype(jnp.float32)
    c = oh
    sh = 1
    while sh < P2:
        c = c + jnp.concatenate(
            [jnp.zeros((E, sh), jnp.float32), c[:, : P2 - sh]], axis=1)
        sh *= 2
    counts = c[:, P2 - 1 : P2]
    rank = c - oh
    caps = jnp.ceil(counts * (1.0 / BLK)) * BLK
    ic = caps
    sh = 1
    while sh < E:
        ic = ic + jnp.concatenate(
            [jnp.zeros((sh, 1), jnp.float32), ic[: E - sh]], axis=0)
        sh *= 2
    gs = ic - caps
    slot_f = jnp.sum(oh * (gs + rank), axis=0, keepdims=True)
    slot_ref[...] = slot_f.astype(jnp.int32)

    bstart = lax.broadcasted_iota(jnp.int32, (E, NB), 1) * BLK
    be = jnp.sum((bstart >= ic.astype(jnp.int32)).astype(jnp.int32),
                 axis=0, keepdims=True)
    be_ref[...] = jnp.minimum(be, E - 1)


_router = pl.pallas_call(
    _router_body,
    out_shape=(
        jax.ShapeDtypeStruct((1, P2), jnp.float32),
        jax.ShapeDtypeStruct((1, P2), jnp.int32),
        jax.ShapeDtypeStruct((1, NB), jnp.int32),
        jax.ShapeDtypeStruct((1, 1), jnp.float32),
    ),
    out_specs=(
        pl.BlockSpec(memory_space=pltpu.VMEM),
        pl.BlockSpec(memory_space=pltpu.VMEM),
        pl.BlockSpec(memory_space=pltpu.VMEM),
        pl.BlockSpec(memory_space=pltpu.SMEM),
    ),
)


@functools.cache
def _build_kernel():
    return pl.kernel(
        _build_body,
        out_type=(
            jax.ShapeDtypeStruct((P,), jnp.int32),
            jax.ShapeDtypeStruct((P,), jnp.float32),
        ),
        mesh=_sc_mesh(),
        compiler_params=pltpu.CompilerParams(needs_layout_passes=False),
        scratch_types=(
            pltpu.VMEM((P,), jnp.int32),
            pltpu.VMEM((P,), jnp.float32),
            pltpu.VMEM((P2,), jnp.int32),
            pltpu.VMEM((P2,), jnp.int32),
            pltpu.VMEM((P2,), jnp.float32),
        ),
    )


def _build_body(slot_hbm, tok_hbm, wp_hbm, gidx_hbm, wslot_hbm,
                g_v, w_v, slot_v, tok_v, wp_v):
    cid = lax.axis_index("c")
    sid = lax.axis_index("s")

    @pl.when(jnp.logical_and(cid == 0, sid == 0))
    def _():
        pltpu.sync_copy(slot_hbm, slot_v)
        pltpu.sync_copy(tok_hbm, tok_v)
        pltpu.sync_copy(wp_hbm, wp_v)

        def _zero(i, carry):
            g_v[pl.ds(i * LANES, LANES)] = jnp.zeros((LANES,), jnp.int32)
            w_v[pl.ds(i * LANES, LANES)] = jnp.zeros((LANES,), jnp.float32)
            return carry
        lax.fori_loop(0, P // LANES, _zero, 0)

        def _scat(i, carry):
            sl = slot_v[pl.ds(i * LANES, LANES)]
            plsc.store_scatter(g_v, [sl], tok_v[pl.ds(i * LANES, LANES)])
            plsc.store_scatter(w_v, [sl], wp_v[pl.ds(i * LANES, LANES)])
            return carry
        lax.fori_loop(0, P2 // LANES, _scat, 0)

        pltpu.sync_copy(g_v, gidx_hbm)
        pltpu.sync_copy(w_v, wslot_hbm)


_ROWS_W = P // NW
_CHUNK = _ROWS_W // 2


@functools.cache
def _dispatch_kernel():
    return pl.kernel(
        _dispatch_body,
        out_type=jax.ShapeDtypeStruct((P, D), jnp.float32),
        mesh=_sc_mesh(),
        compiler_params=pltpu.CompilerParams(needs_layout_passes=False),
        scratch_types=(
            pltpu.VMEM((_CHUNK,), jnp.int32),
            pltpu.VMEM((_CHUNK, D), jnp.float32),
            pltpu.SemaphoreType.DMA,
        ),
    )


def _dispatch_body(x_hbm, gidx_hbm, xs_hbm, idx_v, rows_v, sem):
    cid = lax.axis_index("c")
    sid = lax.axis_index("s")
    wid = sid * NC + cid
    for ch in range(_ROWS_W // _CHUNK):
        base = wid * _ROWS_W + ch * _CHUNK
        pltpu.sync_copy(gidx_hbm.at[pl.ds(base, _CHUNK)], idx_v)
        pltpu.async_copy(x_hbm.at[idx_v], rows_v, sem).wait()
        pltpu.sync_copy(rows_v, xs_hbm.at[pl.ds(base, _CHUNK)])


def _ffn_body(be_ref, xs_ref, w1_ref, b1_ref, w2_ref, b2_ref, ws_ref, out_ref):
    h = jnp.dot(xs_ref[...], w1_ref[0],
                preferred_element_type=jnp.float32) + b1_ref[0]
    h = 0.5 * h * (1.0 + lax.erf(h * 0.7071067811865476))
    y = jnp.dot(h, w2_ref[0],
                preferred_element_type=jnp.float32) + b2_ref[0]
    out_ref[...] = y * ws_ref[...]


_ffn = pl.pallas_call(
    _ffn_body,
    grid_spec=pltpu.PrefetchScalarGridSpec(
        num_scalar_prefetch=1,
        grid=(NB,),
        in_specs=[
            pl.BlockSpec((BLK, D), lambda i, be: (i, 0)),
            pl.BlockSpec((1, D, F), lambda i, be: (be[i], 0, 0)),
            pl.BlockSpec((1, 1, F), lambda i, be: (be[i], 0, 0)),
            pl.BlockSpec((1, F, D), lambda i, be: (be[i], 0, 0)),
            pl.BlockSpec((1, 1, D), lambda i, be: (be[i], 0, 0)),
            pl.BlockSpec((BLK, 1), lambda i, be: (i, 0)),
        ],
        out_specs=pl.BlockSpec((BLK, D), lambda i, be: (i, 0)),
    ),
    out_shape=jax.ShapeDtypeStruct((P, D), jnp.float32),
    compiler_params=pltpu.CompilerParams(
        dimension_semantics=("arbitrary",)),
)


_TOK_W = N // NW


@functools.cache
def _combine_kernel():
    return pl.kernel(
        _combine_body,
        out_type=jax.ShapeDtypeStruct((N, D), jnp.float32),
        mesh=_sc_mesh(),
        compiler_params=pltpu.CompilerParams(needs_layout_passes=False),
        scratch_types=(
            pltpu.VMEM((_TOK_W,), jnp.int32),
            pltpu.VMEM((_TOK_W,), jnp.int32),
            pltpu.VMEM((_TOK_W, D), jnp.float32),
            pltpu.VMEM((_TOK_W, D), jnp.float32),
            pltpu.SemaphoreType.DMA,
            pltpu.SemaphoreType.DMA,
        ),
    )


def _combine_body(ysw_hbm, s1_hbm, s2_hbm, out_hbm,
                  idx1_v, idx2_v, buf1_v, buf2_v, sem1, sem2):
    cid = lax.axis_index("c")
    sid = lax.axis_index("s")
    wid = sid * NC + cid
    base = wid * _TOK_W
    pltpu.sync_copy(s1_hbm.at[pl.ds(base, _TOK_W)], idx1_v)
    pltpu.sync_copy(s2_hbm.at[pl.ds(base, _TOK_W)], idx2_v)
    cp1 = pltpu.async_copy(ysw_hbm.at[idx1_v], buf1_v, sem1)
    cp2 = pltpu.async_copy(ysw_hbm.at[idx2_v], buf2_v, sem2)
    cp1.wait()
    cp2.wait()

    def _row(r, carry):
        for j in range(D // LANES):
            sl = pl.ds(j * LANES, LANES)
            buf1_v[r, sl] = buf1_v[r, sl] + buf2_v[r, sl]
        return carry
    lax.fori_loop(0, _TOK_W, _row, 0)
    pltpu.sync_copy(buf1_v, out_hbm.at[pl.ds(base, _TOK_W)])


def kernel(x, router_W, router_b, W1, b1, W2, b2):
    x_flat = x.reshape(N, D)
    wpair2d, slot2d, be2d, aux = _router(
        router_W.T, router_b.reshape(E, 1), x_flat.T)
    slot = slot2d.reshape(P2)
    tok = jnp.concatenate(
        [jnp.arange(N, dtype=jnp.int32), jnp.arange(N, dtype=jnp.int32)])
    gidx, wslot = _build_kernel()(slot, tok, wpair2d.reshape(P2))
    xs = _dispatch_kernel()(x_flat, gidx)
    ysw = _ffn(be2d.reshape(NB), xs, W1, b1.reshape(E, 1, F), W2,
               b2.reshape(E, 1, D), wslot.reshape(P, 1))
    out = _combine_kernel()(ysw, slot[:N], slot[N:])
    return out.reshape(x.shape), aux[0, 0]
